# baseline (speedup 1.0000x reference)
import functools

import jax
import jax.numpy as jnp
from jax import lax
from jax.experimental import pallas as pl
from jax.experimental.pallas import tpu as pltpu
from jax.experimental.pallas import tpu_sc as plsc

D = 1398
L = 16
V = 100000
VB = 2048
PADV = 100352
PROWS = PADV // 128


def _tc_matvec(t1t, t2t, w2):
    def body(t1_ref, t2_ref, w_ref, out1_ref, out2_ref):
        w1 = w_ref[0:1, :]
        w2_ = w_ref[1:2, :]
        a1 = jnp.dot(w1, t1_ref[...], preferred_element_type=jnp.float32)
        a2 = jnp.dot(w2_, t2_ref[...], preferred_element_type=jnp.float32)
        out1_ref[...] = a1.reshape(VB // 128, 128)
        out2_ref[...] = a2.reshape(VB // 128, 128)

    return pl.pallas_call(
        body,
        grid=(PADV // VB,),
        in_specs=[
            pl.BlockSpec((D, VB), lambda v: (0, v)),
            pl.BlockSpec((D, VB), lambda v: (0, v)),
            pl.BlockSpec((2, D), lambda v: (0, 0)),
        ],
        out_specs=[
            pl.BlockSpec((VB // 128, 128), lambda v: (v, 0)),
            pl.BlockSpec((VB // 128, 128), lambda v: (v, 0)),
        ],
        out_shape=[
            jax.ShapeDtypeStruct((PROWS, 128), jnp.float32),
            jax.ShapeDtypeStruct((PROWS, 128), jnp.float32),
        ],
    )(t1t, t2t, w2)


@functools.lru_cache(maxsize=None)
def _build_probe(NC, NS):
    NW = NC * NS
    mesh = plsc.VectorSubcoreMesh(core_axis_name="c", subcore_axis_name="s")

    @functools.partial(
        pl.kernel,
        mesh=mesh,
        compiler_params=pltpu.CompilerParams(needs_layout_passes=False),
        out_type=jax.ShapeDtypeStruct((NW, 16), jnp.float32),
        scratch_types=[
            pltpu.VMEM((2, 256, 128), jnp.float32),
            pltpu.SemaphoreType.DMA,
            pltpu.SemaphoreType.DMA,
        ],
    )
    def k(t_hbm, out_hbm, buf, s0, s1):
        wid = lax.axis_index("s") * NC + lax.axis_index("c")
        sems = (s0, s1)
        NIT = 64

        def cp(i, b):
            return pltpu.make_async_copy(
                t_hbm.at[pl.ds(256 * (i % 5), 256),
                         pl.ds(wid * 2560 + (i % 8) * 128, 128)],
                buf.at[b], sems[b])

        cp(0, 0).start()
        def body(i, carry):
            for b in range(2):
                g = i * 2 + b
                @pl.when(g + 1 < NIT)
                def _():
                    cp(g + 1, 1 - b).start()
                cp(g, b).wait()
            return carry
        lax.fori_loop(0, NIT // 2, body, 0)
        pltpu.sync_copy(buf.at[0, 0, pl.ds(0, 16)], out_hbm.at[wid])

    return k


@functools.lru_cache(maxsize=None)
def _build_sc(B, NC, NS):
    NW = NC * NS
    BPW = B // NW
    NCH = BPW // L
    mesh = plsc.VectorSubcoreMesh(core_axis_name="c", subcore_axis_name="s")

    @functools.partial(
        pl.kernel,
        mesh=mesh,
        compiler_params=pltpu.CompilerParams(needs_layout_passes=False),
        out_type=jax.ShapeDtypeStruct((NW, BPW), jnp.float32),
        scratch_types=[
            pltpu.VMEM((BPW,), jnp.int32),
            pltpu.VMEM((BPW,), jnp.int32),
            pltpu.VMEM((BPW,), jnp.int32),
            pltpu.VMEM((BPW,), jnp.int32),
            pltpu.VMEM((BPW,), jnp.float32),
            pltpu.VMEM((2 * L,), jnp.float32),
            pltpu.VMEM((BPW, 128), jnp.float32),
            pltpu.VMEM((BPW, 128), jnp.float32),
            pltpu.VMEM((BPW,), jnp.float32),
            pltpu.SemaphoreType.DMA,
            pltpu.SemaphoreType.DMA,
        ],
    )
    def k(idx1_hbm, idx2_hbm, sim_hbm, wb_hbm, p1_hbm, p2_hbm, out_hbm,
          idx1_v, idx2_v, row1_v, row2_v, sim_v, wb_v, rows1, rows2, out_v,
          sem1, sem2):
        wid = lax.axis_index("s") * NC + lax.axis_index("c")
        pltpu.sync_copy(idx1_hbm.at[wid], idx1_v)
        pltpu.sync_copy(idx2_hbm.at[wid], idx2_v)
        pltpu.sync_copy(sim_hbm.at[wid], sim_v)
        pltpu.sync_copy(wb_hbm, wb_v)
        wlast = wb_v[pl.ds(0, L)]
        bias = wb_v[pl.ds(L, L)]

        for c in range(NCH):
            row1_v[pl.ds(c * L, L)] = idx1_v[pl.ds(c * L, L)] >> 7
            row2_v[pl.ds(c * L, L)] = idx2_v[pl.ds(c * L, L)] >> 7
        c1 = pltpu.make_async_copy(p1_hbm.at[row1_v], rows1, sem1)
        c2 = pltpu.make_async_copy(p2_hbm.at[row2_v], rows2, sem2)
        c1.start()
        c2.start()
        c1.wait()
        c2.wait()

        iota = lax.iota(jnp.int32, L)
        for c in range(NCH):
            l1 = idx1_v[pl.ds(c * L, L)] & 127
            l2 = idx2_v[pl.ds(c * L, L)] & 127
            g1 = plsc.load_gather(rows1, [iota + c * L, l1])
            g2 = plsc.load_gather(rows2, [iota + c * L, l2])
            x = g1 + g2 + sim_v[pl.ds(c * L, L)] * wlast + bias
            out_v[pl.ds(c * L, L)] = 1.0 / (1.0 + jnp.exp(-x))

        pltpu.sync_copy(out_v, out_hbm.at[wid])

    return k


def kernel(input1, input2, emb_scenario, emb_law, W_fc, b_fc, similarities):
    B = input1.shape[0]
    info = plsc.get_sparse_core_info()
    NC, NS = info.num_cores, info.num_subcores
    NW = NC * NS
    BPW = B // NW
    wf = W_fc.reshape(-1).astype(jnp.float32)
    w2 = jnp.stack([wf[:D], wf[D:2 * D]])
    p1, p2 = _tc_matvec(emb_scenario.T, emb_law.T, w2)
    idx1 = input1.astype(jnp.int32).reshape(NW, BPW)
    idx2 = input2.astype(jnp.int32).reshape(NW, BPW)
    sim = similarities.astype(jnp.float32).reshape(NW, BPW)
    wb = jnp.concatenate([
        jnp.broadcast_to(wf[2 * D], (L,)),
        jnp.broadcast_to(b_fc.reshape(-1).astype(jnp.float32)[0], (L,)),
    ])
    dummy = _build_probe(NC, NS)(emb_scenario.T)
    out = _build_sc(B, NC, NS)(idx1, idx2, sim, wb, p1, p2)
    return (out + dummy.reshape(-1)[0] * 0.0).reshape(B, 1)


# speedup vs baseline: 1.2571x; 1.2571x over previous
import functools

import jax
import jax.numpy as jnp
from jax import lax
from jax.experimental import pallas as pl
from jax.experimental.pallas import tpu as pltpu
from jax.experimental.pallas import tpu_sc as plsc

D = 1398
L = 16
V = 100000
VB = 2048
PADV = 100352
PROWS = PADV // 128


def _tc_matvec(t1t, t2t, w2):
    def body(t1_ref, t2_ref, w_ref, out1_ref, out2_ref):
        w1 = w_ref[0:1, :]
        w2_ = w_ref[1:2, :]
        a1 = jnp.dot(w1, t1_ref[...], preferred_element_type=jnp.float32)
        a2 = jnp.dot(w2_, t2_ref[...], preferred_element_type=jnp.float32)
        out1_ref[...] = a1.reshape(VB // 128, 128)
        out2_ref[...] = a2.reshape(VB // 128, 128)

    return pl.pallas_call(
        body,
        grid=(PADV // VB,),
        in_specs=[
            pl.BlockSpec((D, VB), lambda v: (0, v)),
            pl.BlockSpec((D, VB), lambda v: (0, v)),
            pl.BlockSpec((2, D), lambda v: (0, 0)),
        ],
        out_specs=[
            pl.BlockSpec((VB // 128, 128), lambda v: (v, 0)),
            pl.BlockSpec((VB // 128, 128), lambda v: (v, 0)),
        ],
        out_shape=[
            jax.ShapeDtypeStruct((PROWS, 128), jnp.float32),
            jax.ShapeDtypeStruct((PROWS, 128), jnp.float32),
        ],
    )(t1t, t2t, w2)


@functools.lru_cache(maxsize=None)
def _build_sc(B, NC, NS):
    NW = NC * NS
    BPW = B // NW
    NCH = BPW // L
    mesh = plsc.VectorSubcoreMesh(core_axis_name="c", subcore_axis_name="s")

    @functools.partial(
        pl.kernel,
        mesh=mesh,
        compiler_params=pltpu.CompilerParams(needs_layout_passes=False),
        out_type=jax.ShapeDtypeStruct((NW, BPW), jnp.float32),
        scratch_types=[
            pltpu.VMEM((BPW,), jnp.int32),
            pltpu.VMEM((BPW,), jnp.int32),
            pltpu.VMEM((BPW,), jnp.int32),
            pltpu.VMEM((BPW,), jnp.int32),
            pltpu.VMEM((BPW,), jnp.float32),
            pltpu.VMEM((2 * L,), jnp.float32),
            pltpu.VMEM((BPW, 128), jnp.float32),
            pltpu.VMEM((BPW, 128), jnp.float32),
            pltpu.VMEM((BPW,), jnp.float32),
            pltpu.SemaphoreType.DMA,
            pltpu.SemaphoreType.DMA,
        ],
    )
    def k(idx1_hbm, idx2_hbm, sim_hbm, wb_hbm, p1_hbm, p2_hbm, out_hbm,
          idx1_v, idx2_v, row1_v, row2_v, sim_v, wb_v, rows1, rows2, out_v,
          sem1, sem2):
        wid = lax.axis_index("s") * NC + lax.axis_index("c")
        pltpu.sync_copy(idx1_hbm.at[wid], idx1_v)
        pltpu.sync_copy(idx2_hbm.at[wid], idx2_v)
        pltpu.sync_copy(sim_hbm.at[wid], sim_v)
        pltpu.sync_copy(wb_hbm, wb_v)
        wlast = wb_v[pl.ds(0, L)]
        bias = wb_v[pl.ds(L, L)]

        for c in range(NCH):
            row1_v[pl.ds(c * L, L)] = idx1_v[pl.ds(c * L, L)] >> 7
            row2_v[pl.ds(c * L, L)] = idx2_v[pl.ds(c * L, L)] >> 7
        c1 = pltpu.make_async_copy(p1_hbm.at[row1_v], rows1, sem1)
        c2 = pltpu.make_async_copy(p2_hbm.at[row2_v], rows2, sem2)
        c1.start()
        c2.start()
        c1.wait()
        c2.wait()

        iota = lax.iota(jnp.int32, L)
        for c in range(NCH):
            l1 = idx1_v[pl.ds(c * L, L)] & 127
            l2 = idx2_v[pl.ds(c * L, L)] & 127
            g1 = plsc.load_gather(rows1, [iota + c * L, l1])
            g2 = plsc.load_gather(rows2, [iota + c * L, l2])
            x = g1 + g2 + sim_v[pl.ds(c * L, L)] * wlast + bias
            out_v[pl.ds(c * L, L)] = 1.0 / (1.0 + jnp.exp(-x))

        pltpu.sync_copy(out_v, out_hbm.at[wid])

    return k


def kernel(input1, input2, emb_scenario, emb_law, W_fc, b_fc, similarities):
    B = input1.shape[0]
    info = plsc.get_sparse_core_info()
    NC, NS = info.num_cores, info.num_subcores
    NW = NC * NS
    BPW = B // NW
    wf = W_fc.reshape(-1).astype(jnp.float32)
    w2 = jnp.stack([wf[:D], wf[D:2 * D]])
    p1, p2 = _tc_matvec(emb_scenario.T, emb_law.T, w2)
    idx1 = input1.astype(jnp.int32).reshape(NW, BPW)
    idx2 = input2.astype(jnp.int32).reshape(NW, BPW)
    sim = similarities.astype(jnp.float32).reshape(NW, BPW)
    wb = jnp.concatenate([
        jnp.broadcast_to(wf[2 * D], (L,)),
        jnp.broadcast_to(b_fc.reshape(-1).astype(jnp.float32)[0], (L,)),
    ])
    out = _build_sc(B, NC, NS)(idx1, idx2, sim, wb, p1, p2)
    return out.reshape(B, 1)
